# Initial kernel scaffold; baseline (speedup 1.0000x reference)
import jax
import jax.numpy as jnp
from jax.experimental import pallas as pl


def kernel(query, refer_bbox, value, W_off, b_off, W_att, b_att, W_val, b_val, W_out, b_out):
    raise NotImplementedError("write your pallas kernel here")


# speedup vs baseline: 94.8116x; 94.8116x over previous
import functools
import math

import numpy as np
import jax
import jax.numpy as jnp
from jax import lax
from jax.experimental import pallas as pl
from jax.experimental.pallas import tpu as pltpu
from jax.experimental.pallas import tpu_sc as plsc

D_MODEL = 256
N_LEVELS = 4
N_HEADS = 8
N_POINTS = 4
VALUE_SHAPES = ((64, 64), (32, 32), (16, 16), (8, 8))
LEVEL_BASE = (0, 4096, 5120, 5376)
BS = 4
LEN_Q = 5440
LEN_V = 5440
N = BS * LEN_Q
BLK = 680
NBLK = N // BLK

NC, NS = 2, 16
NW = NC * NS
QPW = N // NW
QCH = 4
NCH = QPW // QCH


def _build_consts():
    j = np.arange(128)
    l = j // 32
    h = (j % 32) // 4
    p = j % 4
    Wv = np.array([s[1] for s in VALUE_SHAPES], np.float32)[l]
    Hv = np.array([s[0] for s in VALUE_SHAPES], np.float32)[l]
    base = np.array(LEVEL_BASE, np.float32)[l]
    hv = h.astype(np.float32)
    perm_off = np.zeros(256, np.int64)
    for xy in range(2):
        perm_off[xy * 128 + j] = ((h * 4 + l) * 4 + p) * 2 + xy
    perm_att = (h * 16 + l * 4 + p).astype(np.int64)
    RX = np.zeros((8, 128), np.float32)
    RY = np.zeros((8, 128), np.float32)
    RX[2 * l, j] = Wv
    RY[2 * l + 1, j] = Hv
    G2 = (h[:, None] == h[None, :]).astype(np.float32)
    CL = np.stack([Wv, Hv, base, hv], 0)
    return perm_off, perm_att, RX, RY, G2, CL


_PERM_OFF, _PERM_ATT, _RX, _RY, _G2, _CL = _build_consts()


def _prep_body(q_ref, rb_ref, val_ref, woff_ref, boff_ref, watt_ref, batt_ref,
               wval_ref, bval_ref, rx_ref, ry_ref, g2_ref, cl_ref,
               idx_ref, w_ref, vp_ref):
    f32 = jnp.float32
    q = q_ref[...]
    vp_ref[...] = (
        jnp.dot(val_ref[...], wval_ref[...], preferred_element_type=f32, precision="highest")
        + bval_ref[...]
    )
    so = jnp.dot(q, woff_ref[...], preferred_element_type=f32, precision="highest") + boff_ref[...]
    logits = jnp.dot(q, watt_ref[...], preferred_element_type=f32, precision="highest") + batt_ref[...]
    e = jnp.exp(logits)
    aw = e / jnp.dot(e, g2_ref[...], preferred_element_type=f32, precision="highest")
    rb = rb_ref[...]
    Wv = cl_ref[0:1, :]
    Hv = cl_ref[1:2, :]
    base = cl_ref[2:3, :]
    hv = cl_ref[3:4, :]
    px = jnp.dot(rb, rx_ref[...], preferred_element_type=f32, precision="highest") + so[:, :128] - 0.5
    py = jnp.dot(rb, ry_ref[...], preferred_element_type=f32, precision="highest") + so[:, 128:] - 0.5
    x0 = jnp.floor(px)
    y0 = jnp.floor(py)
    dx = px - x0
    dy = py - y0
    bbase = ((pl.program_id(0) // 8) * (LEN_V * N_HEADS)).astype(jnp.int32)
    for c, (cx, cy, wgt) in enumerate((
            (x0, y0, (1.0 - dx) * (1.0 - dy)),
            (x0 + 1.0, y0, dx * (1.0 - dy)),
            (x0, y0 + 1.0, (1.0 - dx) * dy),
            (x0 + 1.0, y0 + 1.0, dx * dy))):
        valid = ((cx >= 0.0) & (cx < Wv) & (cy >= 0.0) & (cy < Hv)).astype(f32)
        xc = jnp.clip(cx, 0.0, Wv - 1.0)
        yc = jnp.clip(cy, 0.0, Hv - 1.0)
        lin = (yc * Wv + xc + base) * 8.0 + hv
        idx_ref[:, c * 128:(c + 1) * 128] = lin.astype(jnp.int32) + bbase
        w_ref[:, c * 128:(c + 1) * 128] = wgt * valid * aw


def _mm_body(x_ref, w_ref, b_ref, o_ref):
    o_ref[...] = (
        jnp.dot(x_ref[...], w_ref[...], preferred_element_type=jnp.float32, precision="highest")
        + b_ref[...]
    )


def _sc_gather(table_hbm, idx_hbm, w_hbm, out_hbm, idx_v, rows_v, w_v, out_v, sem):
    cid = lax.axis_index("c")
    sid = lax.axis_index("s")
    wid = sid * NC + cid
    q0 = wid * QPW

    @pl.loop(0, NCH)
    def _chunk(ch):
        qg = q0 + ch * QCH
        pltpu.sync_copy(idx_hbm.at[pl.ds(qg * 4, QCH * 4)], idx_v)
        pltpu.sync_copy(w_hbm.at[pl.ds(qg * 512, QCH * 512)],
                        w_v.at[pl.ds(0, QCH * 512)])
        copies = [
            pltpu.async_copy(table_hbm.at[idx_v.at[jj]],
                             rows_v.at[pl.ds(jj * 128, 128)], sem)
            for jj in range(QCH * 4)
        ]
        for cp in copies:
            cp.wait()

        @pl.loop(0, QCH * N_HEADS)
        def _qh(qh):
            q = qh // N_HEADS
            h = qh % N_HEADS
            rbase = q * 512 + h * 4
            acc0 = jnp.zeros((16,), jnp.float32)
            acc1 = jnp.zeros((16,), jnp.float32)
            for u in range(16):
                goff = rbase + (u >> 2) * 128 + (u & 3) * 32
                wvec = w_v[pl.ds(goff, 16)]
                for p in range(4):
                    r = goff + p
                    ws = wvec[p]
                    acc0 = acc0 + ws * rows_v[r, 0:16]
                    acc1 = acc1 + ws * rows_v[r, 16:32]
            out_v[q, pl.ds(h * 32, 16)] = acc0
            out_v[q, pl.ds(h * 32 + 16, 16)] = acc1

        pltpu.sync_copy(out_v, out_hbm.at[pl.ds(qg, QCH)])


def kernel(query, refer_bbox, value, W_off, b_off, W_att, b_att, W_val, b_val,
           W_out, b_out):
    f32 = jnp.float32
    q2 = query.reshape(N, D_MODEL)
    rb = refer_bbox.reshape(N, 8)
    v2 = value.reshape(BS * LEN_V, D_MODEL)
    wt_off = W_off.T[:, _PERM_OFF]
    bt_off = b_off[_PERM_OFF].reshape(1, 256)
    wt_att = W_att.T[:, _PERM_ATT]
    bt_att = b_att[_PERM_ATT].reshape(1, 128)
    rx = jnp.asarray(_RX)
    ry = jnp.asarray(_RY)
    g2 = jnp.asarray(_G2)
    cl = jnp.asarray(_CL)

    row_spec = lambda cols: pl.BlockSpec((BLK, cols), lambda i: (i, 0))
    const_spec = lambda shp: pl.BlockSpec(shp, lambda i: (0,) * len(shp))

    idx, w, vp = pl.pallas_call(
        _prep_body,
        grid=(NBLK,),
        in_specs=[
            row_spec(256), row_spec(8), row_spec(256),
            const_spec((256, 256)), const_spec((1, 256)),
            const_spec((256, 128)), const_spec((1, 128)),
            const_spec((256, 256)), const_spec((1, 256)),
            const_spec((8, 128)), const_spec((8, 128)),
            const_spec((128, 128)), const_spec((4, 128)),
        ],
        out_specs=[row_spec(512), row_spec(512), row_spec(256)],
        out_shape=[
            jax.ShapeDtypeStruct((N, 512), jnp.int32),
            jax.ShapeDtypeStruct((N, 512), f32),
            jax.ShapeDtypeStruct((N, 256), f32),
        ],
    )(q2, rb, v2, wt_off, bt_off, wt_att, bt_att, W_val.T,
      b_val.reshape(1, 256), rx, ry, g2, cl)

    table = vp.reshape(BS * LEN_V * N_HEADS, 32)
    idx_r = idx.reshape(N * 4, 128)
    w_r = w.reshape(N * 512)

    mesh = plsc.VectorSubcoreMesh(core_axis_name="c", subcore_axis_name="s")
    sampled = pl.kernel(
        _sc_gather,
        mesh=mesh,
        compiler_params=pltpu.CompilerParams(use_tc_tiling_on_sc=False),
        out_type=jax.ShapeDtypeStruct((N, 256), f32),
        scratch_types=[
            pltpu.VMEM((QCH * 4, 128), jnp.int32),
            pltpu.VMEM((QCH * 512, 32), f32),
            pltpu.VMEM((QCH * 512 + 16,), f32),
            pltpu.VMEM((QCH, 256), f32),
            pltpu.SemaphoreType.DMA,
        ],
    )(table, idx_r, w_r)

    out = pl.pallas_call(
        _mm_body,
        grid=(NBLK,),
        in_specs=[row_spec(256), const_spec((256, 256)), const_spec((1, 256))],
        out_specs=row_spec(256),
        out_shape=jax.ShapeDtypeStruct((N, 256), f32),
    )(sampled, W_out.T, b_out.reshape(1, 256))

    return out.reshape(BS, LEN_Q, D_MODEL)


# speedup vs baseline: 173.6495x; 1.8315x over previous
import functools
import math

import numpy as np
import jax
import jax.numpy as jnp
from jax import lax
from jax.experimental import pallas as pl
from jax.experimental.pallas import tpu as pltpu
from jax.experimental.pallas import tpu_sc as plsc

D_MODEL = 256
N_LEVELS = 4
N_HEADS = 8
N_POINTS = 4
VALUE_SHAPES = ((64, 64), (32, 32), (16, 16), (8, 8))
LEVEL_BASE = (0, 4096, 5120, 5376)
BS = 4
LEN_Q = 5440
LEN_V = 5440
N = BS * LEN_Q
BLK = 1088
NBLK = N // BLK
QB_PER_B = LEN_Q // BLK

NC, NS = 2, 16
NW = NC * NS
QPW = N // NW
QCH = 4
NCH = QPW // QCH


def _build_consts():
    j = np.arange(128)
    l = j // 32
    h = (j % 32) // 4
    p = j % 4
    Wv = np.array([s[1] for s in VALUE_SHAPES], np.float32)[l]
    Hv = np.array([s[0] for s in VALUE_SHAPES], np.float32)[l]
    base = np.array(LEVEL_BASE, np.float32)[l]
    hv = h.astype(np.float32)
    perm_off = np.zeros(256, np.int64)
    for xy in range(2):
        perm_off[xy * 128 + j] = ((h * 4 + l) * 4 + p) * 2 + xy
    perm_att = (h * 16 + l * 4 + p).astype(np.int64)
    RX = np.zeros((8, 128), np.float32)
    RY = np.zeros((8, 128), np.float32)
    RX[2 * l, j] = Wv
    RY[2 * l + 1, j] = Hv
    G2 = (h[:, None] == h[None, :]).astype(np.float32)
    CL = np.stack([Wv, Hv, base, hv], 0)
    colp = np.zeros(256, np.int64)
    for hh in range(8):
        for i in range(16):
            for s in range(2):
                colp[hh * 32 + 2 * i + s] = hh * 32 + i + 16 * s
    return perm_off, perm_att, RX, RY, G2, CL, colp


_PERM_OFF, _PERM_ATT, _RX, _RY, _G2, _CL, _COLP = _build_consts()


def _prep_body(q_ref, rb_ref, val_ref, woff_ref, boff_ref, watt_ref, batt_ref,
               wval_ref, bval_ref, rx_ref, ry_ref, g2_ref, cl_ref,
               idx_ref, w_ref, vp_ref):
    f32 = jnp.float32
    q = q_ref[...]
    vp_ref[...] = (
        jnp.dot(val_ref[...], wval_ref[...], preferred_element_type=f32, precision="highest")
        + bval_ref[...]
    ).astype(jnp.bfloat16)
    so = jnp.dot(q, woff_ref[...], preferred_element_type=f32, precision="highest") + boff_ref[...]
    logits = jnp.dot(q, watt_ref[...], preferred_element_type=f32, precision="highest") + batt_ref[...]
    e = jnp.exp(logits)
    aw = e / jnp.dot(e, g2_ref[...], preferred_element_type=f32, precision="highest")
    rb = rb_ref[...]
    Wv = cl_ref[0:1, :]
    Hv = cl_ref[1:2, :]
    base = cl_ref[2:3, :]
    hv = cl_ref[3:4, :]
    px = jnp.dot(rb, rx_ref[...], preferred_element_type=f32, precision="highest") + so[:, :128] - 0.5
    py = jnp.dot(rb, ry_ref[...], preferred_element_type=f32, precision="highest") + so[:, 128:] - 0.5
    x0 = jnp.floor(px)
    y0 = jnp.floor(py)
    dx = px - x0
    dy = py - y0
    bbase = ((pl.program_id(0) // QB_PER_B) * (LEN_V * N_HEADS)).astype(jnp.int32)
    for c, (cx, cy, wgt) in enumerate((
            (x0, y0, (1.0 - dx) * (1.0 - dy)),
            (x0 + 1.0, y0, dx * (1.0 - dy)),
            (x0, y0 + 1.0, (1.0 - dx) * dy),
            (x0 + 1.0, y0 + 1.0, dx * dy))):
        valid = ((cx >= 0.0) & (cx < Wv) & (cy >= 0.0) & (cy < Hv)).astype(f32)
        xc = jnp.clip(cx, 0.0, Wv - 1.0)
        yc = jnp.clip(cy, 0.0, Hv - 1.0)
        lin = (yc * Wv + xc + base) * 8.0 + hv
        idx_ref[:, c * 128:(c + 1) * 128] = lin.astype(jnp.int32) + bbase
        w_ref[:, c * 128:(c + 1) * 128] = wgt * valid * aw


def _mm_body(x_ref, w_ref, b_ref, o_ref):
    o_ref[...] = (
        jnp.dot(x_ref[...], w_ref[...], preferred_element_type=jnp.float32, precision="highest")
        + b_ref[...]
    )


def _sc_gather(table_hbm, idx_hbm, w_hbm, out_hbm,
               idx_v0, idx_v1, rows_v0, rows_v1, w_v0, w_v1, out_v0, out_v1,
               sem_i0, sem_i1, sem_w0, sem_w1, sem_g0, sem_g1, sem_o0, sem_o1):
    cid = lax.axis_index("c")
    sid = lax.axis_index("s")
    wid = sid * NC + cid
    q0 = wid * QPW

    idx_v = (idx_v0, idx_v1)
    rows_v = (rows_v0, rows_v1)
    w_v = (w_v0, w_v1)
    out_v = (out_v0, out_v1)
    sem_i = (sem_i0, sem_i1)
    sem_w = (sem_w0, sem_w1)
    sem_g = (sem_g0, sem_g1)
    sem_o = (sem_o0, sem_o1)
    NR = QCH * 512

    def issue_idx(ch, b):
        qg = q0 + ch * QCH
        pltpu.async_copy(idx_hbm.at[pl.ds(qg * 4, QCH * 4)], idx_v[b], sem_i[b])

    def issue_w(ch, b):
        qg = q0 + ch * QCH
        pltpu.async_copy(w_hbm.at[pl.ds(qg * 512, NR)],
                         w_v[b].at[pl.ds(0, NR)], sem_w[b])

    def wait_idx(b):
        pltpu.make_async_copy(idx_hbm.at[pl.ds(0, QCH * 4)], idx_v[b],
                              sem_i[b]).wait()

    def wait_w(b):
        pltpu.make_async_copy(w_hbm.at[pl.ds(0, NR)],
                              w_v[b].at[pl.ds(0, NR)], sem_w[b]).wait()

    def issue_gathers(b):
        for jj in range(QCH * 4):
            pltpu.async_copy(table_hbm.at[idx_v[b].at[jj]],
                             rows_v[b].at[pl.ds(jj * 128, 128)], sem_g[b])

    def wait_gathers(b):
        pltpu.make_async_copy(table_hbm.at[pl.ds(0, NR)], rows_v[b],
                              sem_g[b]).wait()

    def issue_out(ch, b):
        qg = q0 + ch * QCH
        pltpu.async_copy(out_v[b], out_hbm.at[pl.ds(qg, QCH)], sem_o[b])

    def wait_out(b):
        pltpu.make_async_copy(out_v[b], out_hbm.at[pl.ds(0, QCH)],
                              sem_o[b]).wait()

    def compute(b):
        @pl.loop(0, QCH * N_HEADS)
        def _qh(qh):
            q = qh // N_HEADS
            h = qh % N_HEADS
            rbase = q * 512 + h * 4
            acc0 = jnp.zeros((16,), jnp.float32)
            acc1 = jnp.zeros((16,), jnp.float32)
            for u in range(16):
                goff = rbase + (u >> 2) * 128 + (u & 3) * 32
                wvec = w_v[b][pl.ds(goff, 16)]
                for p in range(4):
                    row = rows_v[b][goff + p, 0:32]
                    lo, hi = plsc.unpack(row, format=plsc.PackFormat.INTERLEAVED,
                                         preferred_element_type=jnp.float32)
                    ws = wvec[p]
                    acc0 = acc0 + ws * lo
                    acc1 = acc1 + ws * hi
            out_v[b][q, pl.ds(h * 32, 16)] = acc0
            out_v[b][q, pl.ds(h * 32 + 16, 16)] = acc1

    def half(ch, a):
        nb = 1 - a
        @pl.when(ch + 1 < NCH)
        def _():
            wait_idx(nb)
            issue_gathers(nb)
        wait_gathers(a)

        @pl.when(ch + 2 < NCH)
        def _():
            issue_idx(ch + 2, a)

        @pl.when(ch >= 2)
        def _():
            wait_out(a)
        wait_w(a)
        compute(a)
        issue_out(ch, a)

        @pl.when(ch + 2 < NCH)
        def _():
            issue_w(ch + 2, a)

    issue_idx(0, 0)
    issue_w(0, 0)
    wait_idx(0)
    issue_gathers(0)
    issue_idx(1, 1)
    issue_w(1, 1)

    @pl.loop(0, NCH // 2)
    def _pair(t):
        half(2 * t, 0)
        half(2 * t + 1, 1)

    wait_out(0)
    wait_out(1)


def kernel(query, refer_bbox, value, W_off, b_off, W_att, b_att, W_val, b_val,
           W_out, b_out):
    f32 = jnp.float32
    q2 = query.reshape(N, D_MODEL)
    rb = refer_bbox.reshape(N, 8)
    v2 = value.reshape(BS * LEN_V, D_MODEL)
    wt_off = W_off.T[:, _PERM_OFF]
    bt_off = b_off[_PERM_OFF].reshape(1, 256)
    wt_att = W_att.T[:, _PERM_ATT]
    bt_att = b_att[_PERM_ATT].reshape(1, 128)
    rx = jnp.asarray(_RX)
    ry = jnp.asarray(_RY)
    g2 = jnp.asarray(_G2)
    cl = jnp.asarray(_CL)

    row_spec = lambda cols: pl.BlockSpec((BLK, cols), lambda i: (i, 0))
    const_spec = lambda shp: pl.BlockSpec(shp, lambda i: (0,) * len(shp))

    idx, w, vp = pl.pallas_call(
        _prep_body,
        grid=(NBLK,),
        in_specs=[
            row_spec(256), row_spec(8), row_spec(256),
            const_spec((256, 256)), const_spec((1, 256)),
            const_spec((256, 128)), const_spec((1, 128)),
            const_spec((256, 256)), const_spec((1, 256)),
            const_spec((8, 128)), const_spec((8, 128)),
            const_spec((128, 128)), const_spec((4, 128)),
        ],
        out_specs=[row_spec(512), row_spec(512), row_spec(256)],
        out_shape=[
            jax.ShapeDtypeStruct((N, 512), jnp.int32),
            jax.ShapeDtypeStruct((N, 512), f32),
            jax.ShapeDtypeStruct((N, 256), jnp.bfloat16),
        ],
    )(q2, rb, v2, wt_off, bt_off, wt_att, bt_att, W_val.T[:, _COLP],
      b_val[_COLP].reshape(1, 256), rx, ry, g2, cl)

    table = vp.reshape(BS * LEN_V * N_HEADS, 32)
    idx_r = idx.reshape(N * 4, 128)
    w_r = w.reshape(N * 512)

    mesh = plsc.VectorSubcoreMesh(core_axis_name="c", subcore_axis_name="s")
    sampled = pl.kernel(
        _sc_gather,
        mesh=mesh,
        compiler_params=pltpu.CompilerParams(use_tc_tiling_on_sc=False,
                                             needs_layout_passes=False),
        out_type=jax.ShapeDtypeStruct((N, 256), f32),
        scratch_types=[
            pltpu.VMEM((QCH * 4, 128), jnp.int32),
            pltpu.VMEM((QCH * 4, 128), jnp.int32),
            pltpu.VMEM((QCH * 512, 32), jnp.bfloat16),
            pltpu.VMEM((QCH * 512, 32), jnp.bfloat16),
            pltpu.VMEM((QCH * 512 + 16,), f32),
            pltpu.VMEM((QCH * 512 + 16,), f32),
            pltpu.VMEM((QCH, 256), f32),
            pltpu.VMEM((QCH, 256), f32),
        ] + [pltpu.SemaphoreType.DMA] * 8,
    )(table, idx_r, w_r)

    out = pl.pallas_call(
        _mm_body,
        grid=(NBLK,),
        in_specs=[row_spec(256), const_spec((256, 256)), const_spec((1, 256))],
        out_specs=row_spec(256),
        out_shape=jax.ShapeDtypeStruct((N, 256), f32),
    )(sampled, W_out.T, b_out.reshape(1, 256))

    return out.reshape(BS, LEN_Q, D_MODEL)


# speedup vs baseline: 191.6927x; 1.1039x over previous
import functools
import math

import numpy as np
import jax
import jax.numpy as jnp
from jax import lax
from jax.experimental import pallas as pl
from jax.experimental.pallas import tpu as pltpu
from jax.experimental.pallas import tpu_sc as plsc

D_MODEL = 256
N_LEVELS = 4
N_HEADS = 8
N_POINTS = 4
VALUE_SHAPES = ((64, 64), (32, 32), (16, 16), (8, 8))
LEVEL_BASE = (0, 4096, 5120, 5376)
BS = 4
LEN_Q = 5440
LEN_V = 5440
N = BS * LEN_Q
BLK = 1088
NBLK = N // BLK
QB_PER_B = LEN_Q // BLK

NC, NS = 2, 16
NW = NC * NS
QPW = N // NW
QCH = 5
NCH = QPW // QCH


def _build_consts():
    j = np.arange(128)
    h = j // 16
    l = (j % 16) // 4
    p = j % 4
    Wv = np.array([s[1] for s in VALUE_SHAPES], np.float32)[l]
    Hv = np.array([s[0] for s in VALUE_SHAPES], np.float32)[l]
    base = np.array(LEVEL_BASE, np.float32)[l]
    hv = h.astype(np.float32)
    perm_off = np.zeros(256, np.int64)
    for xy in range(2):
        perm_off[xy * 128 + j] = ((h * 4 + l) * 4 + p) * 2 + xy
    perm_att = (h * 16 + l * 4 + p).astype(np.int64)
    RX = np.zeros((8, 128), np.float32)
    RY = np.zeros((8, 128), np.float32)
    RX[2 * l, j] = Wv
    RY[2 * l + 1, j] = Hv
    G2 = (h[:, None] == h[None, :]).astype(np.float32)
    CL = np.stack([Wv, Hv, base, hv], 0)
    colp = np.zeros(256, np.int64)
    for hh in range(8):
        for i in range(16):
            for s in range(2):
                colp[hh * 32 + 2 * i + s] = hh * 32 + i + 16 * s
    return perm_off, perm_att, RX, RY, G2, CL, colp


_PERM_OFF, _PERM_ATT, _RX, _RY, _G2, _CL, _COLP = _build_consts()


def _prep_body(q_ref, rb_ref, val_ref, woff_ref, boff_ref, watt_ref, batt_ref,
               wval_ref, bval_ref, rx_ref, ry_ref, g2_ref, cl_ref,
               idx_ref, w_ref, vp_ref):
    f32 = jnp.float32
    q = q_ref[...]
    vp_ref[...] = (
        jnp.dot(val_ref[...], wval_ref[...], preferred_element_type=f32, precision="highest")
        + bval_ref[...]
    ).astype(jnp.bfloat16)
    so = jnp.dot(q, woff_ref[...], preferred_element_type=f32, precision="highest") + boff_ref[...]
    logits = jnp.dot(q, watt_ref[...], preferred_element_type=f32, precision="highest") + batt_ref[...]
    e = jnp.exp(logits)
    aw = e / jnp.dot(e, g2_ref[...], preferred_element_type=f32, precision="highest")
    rb = rb_ref[...]
    Wv = cl_ref[0:1, :]
    Hv = cl_ref[1:2, :]
    base = cl_ref[2:3, :]
    hv = cl_ref[3:4, :]
    px = jnp.dot(rb, rx_ref[...], preferred_element_type=f32, precision="highest") + so[:, :128] - 0.5
    py = jnp.dot(rb, ry_ref[...], preferred_element_type=f32, precision="highest") + so[:, 128:] - 0.5
    x0 = jnp.floor(px)
    y0 = jnp.floor(py)
    dx = px - x0
    dy = py - y0
    bbase = ((pl.program_id(0) // QB_PER_B) * (LEN_V * N_HEADS)).astype(jnp.int32)
    for c, (cx, cy, wgt) in enumerate((
            (x0, y0, (1.0 - dx) * (1.0 - dy)),
            (x0 + 1.0, y0, dx * (1.0 - dy)),
            (x0, y0 + 1.0, (1.0 - dx) * dy),
            (x0 + 1.0, y0 + 1.0, dx * dy))):
        valid = ((cx >= 0.0) & (cx < Wv) & (cy >= 0.0) & (cy < Hv)).astype(f32)
        xc = jnp.clip(cx, 0.0, Wv - 1.0)
        yc = jnp.clip(cy, 0.0, Hv - 1.0)
        lin = (yc * Wv + xc + base) * 8.0 + hv
        idx_ref[c, :, :] = lin.astype(jnp.int32) + bbase
        w_ref[c, :, :] = wgt * valid * aw


def _mm_body(x_ref, w_ref, b_ref, o_ref):
    o_ref[...] = (
        jnp.dot(x_ref[...], w_ref[...], preferred_element_type=jnp.float32, precision="highest")
        + b_ref[...]
    )


def _sc_gather(table_hbm, idx_hbm, w_hbm, out_hbm,
               idx_v0, idx_v1, rows_v0, rows_v1, w_v0, w_v1, out_v0, out_v1,
               sem_i0, sem_i1, sem_w0, sem_w1, sem_g0, sem_g1, sem_o0, sem_o1):
    cid = lax.axis_index("c")
    sid = lax.axis_index("s")
    wid = sid * NC + cid
    q0 = wid * QPW

    idx_v = (idx_v0, idx_v1)
    rows_v = (rows_v0, rows_v1)
    w_v = (w_v0, w_v1)
    out_v = (out_v0, out_v1)
    sem_i = (sem_i0, sem_i1)
    sem_w = (sem_w0, sem_w1)
    sem_g = (sem_g0, sem_g1)
    sem_o = (sem_o0, sem_o1)
    NR = QCH * 512

    def issue_idx(ch, b):
        qg = q0 + ch * QCH
        for c in range(4):
            pltpu.async_copy(idx_hbm.at[c, pl.ds(qg, QCH), :],
                             idx_v[b].at[pl.ds(c * QCH, QCH), :], sem_i[b])

    def issue_w(ch, b):
        qg = q0 + ch * QCH
        for c in range(4):
            pltpu.async_copy(w_hbm.at[c, pl.ds(qg, QCH), :],
                             w_v[b].at[pl.ds(c * QCH, QCH), :], sem_w[b])

    def wait_idx(b):
        pltpu.make_async_copy(idx_hbm.at[0, pl.ds(0, QCH * 4), :], idx_v[b],
                              sem_i[b]).wait()

    def wait_w(b):
        pltpu.make_async_copy(w_hbm.at[0, pl.ds(0, QCH * 4), :], w_v[b],
                              sem_w[b]).wait()

    def issue_gathers(b):
        for jj in range(QCH * 4):
            pltpu.async_copy(table_hbm.at[idx_v[b].at[jj]],
                             rows_v[b].at[pl.ds(jj * 128, 128)], sem_g[b])

    def wait_gathers(b):
        pltpu.make_async_copy(table_hbm.at[pl.ds(0, NR)], rows_v[b],
                              sem_g[b]).wait()

    def issue_out(ch, b):
        qg = q0 + ch * QCH
        pltpu.async_copy(out_v[b], out_hbm.at[pl.ds(qg, QCH)], sem_o[b])

    def wait_out(b):
        pltpu.make_async_copy(out_v[b], out_hbm.at[pl.ds(0, QCH)],
                              sem_o[b]).wait()

    def compute(b):
        @pl.loop(0, QCH * N_HEADS)
        def _qh(qh):
            q = qh // N_HEADS
            h = qh % N_HEADS
            hb = h * 16
            acc0 = jnp.zeros((16,), jnp.float32)
            acc1 = jnp.zeros((16,), jnp.float32)
            for c in range(4):
                row_i = c * QCH + q
                wvec = w_v[b][row_i, pl.ds(hb, 16)]
                rb2 = row_i * 128 + hb
                for k in range(16):
                    row = rows_v[b][rb2 + k, 0:32]
                    lo, hi = plsc.unpack(row, format=plsc.PackFormat.INTERLEAVED,
                                         preferred_element_type=jnp.float32)
                    ws = wvec[k]
                    acc0 = acc0 + ws * lo
                    acc1 = acc1 + ws * hi
            out_v[b][q, pl.ds(h * 32, 16)] = acc0
            out_v[b][q, pl.ds(h * 32 + 16, 16)] = acc1

    def half(ch, a):
        nb = 1 - a
        @pl.when(ch + 1 < NCH)
        def _():
            wait_idx(nb)
            issue_gathers(nb)
        wait_gathers(a)

        @pl.when(ch + 2 < NCH)
        def _():
            issue_idx(ch + 2, a)

        @pl.when(ch >= 2)
        def _():
            wait_out(a)
        wait_w(a)
        compute(a)
        issue_out(ch, a)

        @pl.when(ch + 2 < NCH)
        def _():
            issue_w(ch + 2, a)

    issue_idx(0, 0)
    issue_w(0, 0)
    wait_idx(0)
    issue_gathers(0)
    issue_idx(1, 1)
    issue_w(1, 1)

    @pl.loop(0, NCH // 2)
    def _pair(t):
        half(2 * t, 0)
        half(2 * t + 1, 1)

    wait_out(0)
    wait_out(1)


def kernel(query, refer_bbox, value, W_off, b_off, W_att, b_att, W_val, b_val,
           W_out, b_out):
    f32 = jnp.float32
    q2 = query.reshape(N, D_MODEL)
    rb = refer_bbox.reshape(N, 8)
    v2 = value.reshape(BS * LEN_V, D_MODEL)
    wt_off = W_off.T[:, _PERM_OFF]
    bt_off = b_off[_PERM_OFF].reshape(1, 256)
    wt_att = W_att.T[:, _PERM_ATT]
    bt_att = b_att[_PERM_ATT].reshape(1, 128)
    rx = jnp.asarray(_RX)
    ry = jnp.asarray(_RY)
    g2 = jnp.asarray(_G2)
    cl = jnp.asarray(_CL)

    row_spec = lambda cols: pl.BlockSpec((BLK, cols), lambda i: (i, 0))
    const_spec = lambda shp: pl.BlockSpec(shp, lambda i: (0,) * len(shp))

    idx, w, vp = pl.pallas_call(
        _prep_body,
        grid=(NBLK,),
        in_specs=[
            row_spec(256), row_spec(8), row_spec(256),
            const_spec((256, 256)), const_spec((1, 256)),
            const_spec((256, 128)), const_spec((1, 128)),
            const_spec((256, 256)), const_spec((1, 256)),
            const_spec((8, 128)), const_spec((8, 128)),
            const_spec((128, 128)), const_spec((4, 128)),
        ],
        out_specs=[
            pl.BlockSpec((4, BLK, 128), lambda i: (0, i, 0)),
            pl.BlockSpec((4, BLK, 128), lambda i: (0, i, 0)),
            row_spec(256),
        ],
        out_shape=[
            jax.ShapeDtypeStruct((4, N, 128), jnp.int32),
            jax.ShapeDtypeStruct((4, N, 128), f32),
            jax.ShapeDtypeStruct((N, 256), jnp.bfloat16),
        ],
    )(q2, rb, v2, wt_off, bt_off, wt_att, bt_att, W_val.T[:, _COLP],
      b_val[_COLP].reshape(1, 256), rx, ry, g2, cl)

    table = vp.reshape(BS * LEN_V * N_HEADS, 32)

    mesh = plsc.VectorSubcoreMesh(core_axis_name="c", subcore_axis_name="s")
    sampled = pl.kernel(
        _sc_gather,
        mesh=mesh,
        compiler_params=pltpu.CompilerParams(use_tc_tiling_on_sc=False,
                                             needs_layout_passes=False),
        out_type=jax.ShapeDtypeStruct((N, 256), f32),
        scratch_types=[
            pltpu.VMEM((QCH * 4, 128), jnp.int32),
            pltpu.VMEM((QCH * 4, 128), jnp.int32),
            pltpu.VMEM((QCH * 512, 32), jnp.bfloat16),
            pltpu.VMEM((QCH * 512, 32), jnp.bfloat16),
            pltpu.VMEM((QCH * 4, 128), f32),
            pltpu.VMEM((QCH * 4, 128), f32),
            pltpu.VMEM((QCH, 256), f32),
            pltpu.VMEM((QCH, 256), f32),
        ] + [pltpu.SemaphoreType.DMA] * 8,
    )(table, idx, w)

    out = pl.pallas_call(
        _mm_body,
        grid=(NBLK,),
        in_specs=[row_spec(256), const_spec((256, 256)), const_spec((1, 256))],
        out_specs=row_spec(256),
        out_shape=jax.ShapeDtypeStruct((N, 256), f32),
    )(sampled, W_out.T, b_out.reshape(1, 256))

    return out.reshape(BS, LEN_Q, D_MODEL)
